# Initial kernel scaffold; baseline (speedup 1.0000x reference)
import jax
import jax.numpy as jnp
from jax.experimental import pallas as pl


def kernel(node, edge, edgeIdx, edge2node, g1, b1, g2, b2, We1, be1, We2, be2, Wn1, bn1, Wn2, bn2):
    raise NotImplementedError("write your pallas kernel here")


# speedup vs baseline: 2.1798x; 2.1798x over previous
import jax
import jax.numpy as jnp
from jax import lax
from jax.experimental import pallas as pl
from jax.experimental.pallas import tpu as pltpu
from jax.experimental.pallas import tpu_sc as plsc

_NC = 2
_NS = 16
_NW = _NC * _NS
_D = 128


def _sc_gather(table, idx2d):
    n_rows, d = idx2d.shape
    v_rows, C = table.shape
    R = n_rows // _NW
    mesh = plsc.VectorSubcoreMesh(core_axis_name="c", subcore_axis_name="s",
                                  num_cores=_NC, num_subcores=_NS)

    def body(table_hbm, idx_hbm, out_hbm, idx_v, rows_v, sem):
        wid = lax.axis_index("s") * _NC + lax.axis_index("c")
        pltpu.sync_copy(idx_hbm.at[pl.ds(wid * R, R)], idx_v)

        def step(k, carry):
            r = wid * R + k
            pltpu.async_copy(table_hbm.at[idx_v.at[k]], rows_v, sem).wait()
            pltpu.sync_copy(rows_v, out_hbm.at[pl.ds(r * d, d)])
            return carry

        lax.fori_loop(0, R, step, 0)

    f = pl.kernel(
        body,
        out_type=jax.ShapeDtypeStruct((n_rows * d, C), table.dtype),
        mesh=mesh,
        scratch_types=[
            pltpu.VMEM((R, d), jnp.int32),
            pltpu.VMEM((d, C), jnp.float32),
            pltpu.SemaphoreType.DMA,
        ],
    )
    return f(table, idx2d)


def _sc_scatter(vals, idx2d, n_out):
    n_rows, d = idx2d.shape
    C = vals.shape[1]
    R = n_rows // _NW
    TBL = ((n_out + _NS * 16 - 1) // (_NS * 16)) * (_NS * 16)
    ZR = TBL // _NS
    c_sz = 128
    mesh = plsc.VectorSubcoreMesh(core_axis_name="c", subcore_axis_name="s",
                                  num_cores=_NC, num_subcores=_NS)

    def body(vals_hbm, idx_hbm, out_hbm, idx_v, rows_v, zb, cb, table_sh):
        cid = lax.axis_index("c")
        sid = lax.axis_index("s")
        wid = sid * _NC + cid
        pltpu.sync_copy(idx_hbm.at[pl.ds(wid * R, R)], idx_v)

        for rr in range(16):
            for cc in range(0, C, 16):
                zb[rr, pl.ds(cc, 16)] = jnp.zeros((16,), jnp.float32)

        def zstep(k, carry):
            pltpu.sync_copy(zb, table_sh.at[pl.ds(sid * ZR + k * 16, 16)])
            return carry

        lax.fori_loop(0, ZR // 16, zstep, 0)
        plsc.subcore_barrier()

        def sstep(k, carry):
            r = wid * R + k
            pltpu.sync_copy(vals_hbm.at[pl.ds(r * d, d)], rows_v)
            pltpu.sync_copy(rows_v, table_sh.at[idx_v.at[k]], add=True)
            return carry

        lax.fori_loop(0, R, sstep, 0)
        plsc.subcore_barrier()

        for m in range(ZR // c_sz):
            off = sid * ZR + m * c_sz
            pltpu.sync_copy(table_sh.at[pl.ds(off, c_sz)], cb)
            pltpu.sync_copy(cb, out_hbm.at[cid, pl.ds(off, c_sz)])

    f = pl.kernel(
        body,
        out_type=jax.ShapeDtypeStruct((_NC, TBL, C), jnp.float32),
        mesh=mesh,
        scratch_types=[
            pltpu.VMEM((R, d), jnp.int32),
            pltpu.VMEM((d, C), jnp.float32),
            pltpu.VMEM((16, C), jnp.float32),
            pltpu.VMEM((c_sz, C), jnp.float32),
            pltpu.VMEM_SHARED((TBL, C), jnp.float32),
        ],
    )
    return f(vals, idx2d)


def _round_to_f16(x):
    u = lax.bitcast_convert_type(x, jnp.uint32)
    lsb = (u >> jnp.uint32(13)) & jnp.uint32(1)
    u = (u + jnp.uint32(0x0FFF) + lsb) & jnp.uint32(0xFFFFE000)
    return lax.bitcast_convert_type(u, jnp.float32)


def _tc_edge(edge2, gath2, g1, b1, W1p, b1p, W2p, b2p, e_pad_rows):
    N_e, C = edge2.shape
    E = 2000
    grid = (N_e // E,)

    def body(ed_ref, gt_ref, g_ref, bb_ref, w1_ref, c1_ref, w2_ref, c2_ref,
             e32_ref, eo_ref):
        ed = ed_ref[...]
        gt = gt_ref[...]
        inv_n = 1.0 / (3 * C)
        mu = (jnp.sum(ed, axis=1, keepdims=True)
              + jnp.sum(gt, axis=1, keepdims=True)) * inv_n
        d1 = ed - mu
        d2 = gt - mu
        var = (jnp.sum(d1 * d1, axis=1, keepdims=True)
               + jnp.sum(d2 * d2, axis=1, keepdims=True)) * inv_n
        s = 1.0 / jnp.sqrt(var + 1e-5)
        x = jnp.concatenate([d1, d2], axis=1) * s * g_ref[...] + bb_ref[...]
        h = jnp.dot(x, w1_ref[...], preferred_element_type=jnp.float32) + c1_ref[...]
        h = 0.5 * h * (1.0 + lax.erf(h * 0.7071067811865476))
        e = jnp.dot(h, w2_ref[...], preferred_element_type=jnp.float32) + c2_ref[...]
        e32 = _round_to_f16(e)
        e32_ref[...] = e32
        eo_ref[...] = ed + e32

    return pl.pallas_call(
        body,
        grid=grid,
        in_specs=[
            pl.BlockSpec((E, C), lambda i: (i, 0)),
            pl.BlockSpec((E, 2 * C), lambda i: (i, 0)),
            pl.BlockSpec((1, 3 * C), lambda i: (0, 0)),
            pl.BlockSpec((1, 3 * C), lambda i: (0, 0)),
            pl.BlockSpec((3 * C, C), lambda i: (0, 0)),
            pl.BlockSpec((1, C), lambda i: (0, 0)),
            pl.BlockSpec((C, C), lambda i: (0, 0)),
            pl.BlockSpec((1, C), lambda i: (0, 0)),
        ],
        out_specs=[
            pl.BlockSpec((E, C), lambda i: (i, 0)),
            pl.BlockSpec((E, C), lambda i: (i, 0)),
        ],
        out_shape=[
            jax.ShapeDtypeStruct((e_pad_rows, C), jnp.float32),
            jax.ShapeDtypeStruct((N_e, C), jnp.float32),
        ],
    )(edge2, gath2, g1, b1, W1p, b1p, W2p, b2p)


def _tc_node(node2, aggs, g2, b2, W1p, b1p, W2p, b2p):
    N_n, C = node2.shape
    NB = 1000
    grid = (N_n // NB,)

    def body(nd_ref, a0_ref, a1_ref, g_ref, bb_ref, w1_ref, c1_ref, w2_ref,
             c2_ref, out_ref):
        nd = nd_ref[...]
        ag = a0_ref[0] + a1_ref[0]
        inv_n = 1.0 / (2 * C)
        mu = (jnp.sum(nd, axis=1, keepdims=True)
              + jnp.sum(ag, axis=1, keepdims=True)) * inv_n
        d1 = nd - mu
        d2 = ag - mu
        var = (jnp.sum(d1 * d1, axis=1, keepdims=True)
               + jnp.sum(d2 * d2, axis=1, keepdims=True)) * inv_n
        s = 1.0 / jnp.sqrt(var + 1e-5)
        x = jnp.concatenate([d1, d2], axis=1) * s * g_ref[...] + bb_ref[...]
        h = jnp.dot(x, w1_ref[...], preferred_element_type=jnp.float32) + c1_ref[...]
        h = 0.5 * h * (1.0 + lax.erf(h * 0.7071067811865476))
        o = jnp.dot(h, w2_ref[...], preferred_element_type=jnp.float32) + c2_ref[...]
        out_ref[...] = nd + o

    return pl.pallas_call(
        body,
        grid=grid,
        in_specs=[
            pl.BlockSpec((NB, C), lambda i: (i, 0)),
            pl.BlockSpec((1, NB, C), lambda i: (0, i, 0)),
            pl.BlockSpec((1, NB, C), lambda i: (1, i, 0)),
            pl.BlockSpec((1, 2 * C), lambda i: (0, 0)),
            pl.BlockSpec((1, 2 * C), lambda i: (0, 0)),
            pl.BlockSpec((2 * C, C), lambda i: (0, 0)),
            pl.BlockSpec((1, C), lambda i: (0, 0)),
            pl.BlockSpec((C, C), lambda i: (0, 0)),
            pl.BlockSpec((1, C), lambda i: (0, 0)),
        ],
        out_specs=pl.BlockSpec((NB, C), lambda i: (i, 0)),
        out_shape=jax.ShapeDtypeStruct((N_n, C), jnp.float32),
    )(node2, aggs, aggs, g2, b2, W1p, b1p, W2p, b2p)


def kernel(node, edge, edgeIdx, edge2node, g1, b1, g2, b2,
           We1, be1, We2, be2, Wn1, bn1, Wn2, bn2):
    B, N_n, C = node.shape
    N_e = edge.shape[1]
    H3 = We1.shape[1]
    H2 = Wn1.shape[1]

    node2 = node.reshape(N_n, C)
    edge2 = edge.reshape(N_e, C)

    nf = 2 * N_e
    Rg = ((-(-nf // (_NW * _D)) + 7) // 8) * 8
    pad_g = _NW * _D * Rg - nf
    idx_flat = edgeIdx.astype(jnp.int32).reshape(-1)
    idx2d = jnp.concatenate(
        [idx_flat, jnp.zeros((pad_g,), jnp.int32)]).reshape(_NW * Rg, _D)
    gath = _sc_gather(node2, idx2d)
    gath2 = gath.reshape(-1, 2 * C)

    W1p = jnp.pad(We1, ((0, 0), (0, C - H3)))
    b1p = jnp.pad(be1, (0, C - H3)).reshape(1, C)
    W2p = jnp.pad(We2, ((0, C - H3), (0, 0)))
    b2p = be2.reshape(1, C)

    ns = N_e // _D
    Rs = ((-(-ns // _NW) + 7) // 8) * 8
    e_pad_rows = _NW * Rs * _D
    TBL = ((N_n + _NS * 16 - 1) // (_NS * 16)) * (_NS * 16)

    e32, edge_out = _tc_edge(edge2, gath2, g1.reshape(1, 3 * C),
                             b1.reshape(1, 3 * C), W1p, b1p, W2p, b2p,
                             e_pad_rows)

    pad_s = e_pad_rows - N_e
    e2n = jnp.concatenate(
        [edge2node.astype(jnp.int32),
         jnp.full((pad_s,), TBL - 1, jnp.int32)]).reshape(_NW * Rs, _D)
    aggs = _sc_scatter(e32, e2n, N_n)

    V1p = jnp.pad(Wn1, ((0, 0), (0, C - H2)))
    d1p = jnp.pad(bn1, (0, C - H2)).reshape(1, C)
    V2p = jnp.pad(Wn2, ((0, C - H2), (0, 0)))
    d2p = bn2.reshape(1, C)

    node_out = _tc_node(node2, aggs, g2.reshape(1, 2 * C),
                        b2.reshape(1, 2 * C), V1p, d1p, V2p, d2p)

    return (node_out.reshape(B, N_n, C), edge_out.reshape(B, N_e, C))


# speedup vs baseline: 2.4388x; 1.1188x over previous
import jax
import jax.numpy as jnp
from jax import lax
from jax.experimental import pallas as pl
from jax.experimental.pallas import tpu as pltpu
from jax.experimental.pallas import tpu_sc as plsc

_NC = 2
_NS = 16
_NW = _NC * _NS
_D = 128


def _sc_gather(table, idx2d):
    n_rows, d = idx2d.shape
    v_rows, C = table.shape
    R = n_rows // _NW
    mesh = plsc.VectorSubcoreMesh(core_axis_name="c", subcore_axis_name="s",
                                  num_cores=_NC, num_subcores=_NS)

    NB = 5
    LOOK = 3
    LAG = 2
    assert (R - LAG - LOOK) % NB == 0 and R > NB

    def body(table_hbm, idx_hbm, out_hbm, idx_v, b0, b1, b2, b3, b4,
             gsem, wsem):
        bufs = (b0, b1, b2, b3, b4)
        wid = lax.axis_index("s") * _NC + lax.axis_index("c")
        pltpu.sync_copy(idx_hbm.at[pl.ds(wid * R, R)], idx_v)
        dummy = table_hbm.at[pl.ds(0, d)]

        def g_start(k, b):
            pltpu.async_copy(table_hbm.at[idx_v.at[k]], bufs[b], gsem)

        def step(k, b, do_wbwait, do_prefetch):
            pltpu.make_async_copy(dummy, bufs[b], gsem).wait()
            pltpu.async_copy(
                bufs[b], out_hbm.at[pl.ds((wid * R + k) * d, d)], wsem)
            if do_wbwait:
                pltpu.make_async_copy(dummy, bufs[b], wsem).wait()
            if do_prefetch:
                g_start(k + LOOK, (b + LOOK) % NB)

        for k in range(LOOK):
            g_start(k, k % NB)
        for k in range(LAG):
            step(k, k % NB, False, True)

        def round_(r, carry):
            k0 = LAG + r * NB
            for i in range(NB):
                b = (LAG + i) % NB
                pltpu.make_async_copy(dummy, bufs[b], gsem).wait()
                pltpu.async_copy(
                    bufs[b],
                    out_hbm.at[pl.ds((wid * R + k0 + i) * d, d)], wsem)
                bl = (LAG + i - LAG) % NB
                pltpu.make_async_copy(dummy, bufs[bl], wsem).wait()
                g_start(k0 + i + LOOK, (LAG + i + LOOK) % NB)
            return carry

        n_rounds = (R - LAG - LOOK) // NB
        lax.fori_loop(0, n_rounds, round_, 0)
        for k in range(LAG + n_rounds * NB, R):
            b = k % NB
            pltpu.make_async_copy(dummy, bufs[b], gsem).wait()
            pltpu.async_copy(
                bufs[b], out_hbm.at[pl.ds((wid * R + k) * d, d)], wsem)
            pltpu.make_async_copy(dummy, bufs[(k - LAG) % NB], wsem).wait()
        for k in range(R - LAG, R):
            pltpu.make_async_copy(dummy, bufs[k % NB], wsem).wait()

    f = pl.kernel(
        body,
        out_type=jax.ShapeDtypeStruct((n_rows * d, C), table.dtype),
        mesh=mesh,
        scratch_types=[
            pltpu.VMEM((R, d), jnp.int32),
            pltpu.VMEM((d, C), jnp.float32),
            pltpu.VMEM((d, C), jnp.float32),
            pltpu.VMEM((d, C), jnp.float32),
            pltpu.VMEM((d, C), jnp.float32),
            pltpu.VMEM((d, C), jnp.float32),
            pltpu.SemaphoreType.DMA,
            pltpu.SemaphoreType.DMA,
        ],
    )
    return f(table, idx2d)


def _sc_scatter(vals, idx2d, n_out):
    n_rows, d = idx2d.shape
    C = vals.shape[1]
    R = n_rows // _NW
    TBL = ((n_out + _NS * 16 - 1) // (_NS * 16)) * (_NS * 16)
    ZR = TBL // _NS
    c_sz = 128
    mesh = plsc.VectorSubcoreMesh(core_axis_name="c", subcore_axis_name="s",
                                  num_cores=_NC, num_subcores=_NS)

    assert R % 2 == 0

    def body(vals_hbm, idx_hbm, out_hbm, idx_v, v0, v1, zb, table_sh,
             lsem):
        cid = lax.axis_index("c")
        sid = lax.axis_index("s")
        wid = sid * _NC + cid
        vb = (v0, v1)
        dummy = vals_hbm.at[pl.ds(0, d)]

        for rr in range(16):
            for cc in range(0, C, 16):
                zb[rr, pl.ds(cc, 16)] = jnp.zeros((16,), jnp.float32)
        def zfire(k, carry):
            pltpu.async_copy(zb, table_sh.at[pl.ds(sid * ZR + k * 16, 16)],
                             lsem)
            return carry

        def zdrain(k, carry):
            pltpu.make_async_copy(vals_hbm.at[pl.ds(0, 16)], zb, lsem).wait()
            return carry

        lax.fori_loop(0, ZR // 16, zfire, 0)
        pltpu.sync_copy(idx_hbm.at[pl.ds(wid * R, R)], idx_v)
        lax.fori_loop(0, ZR // 16, zdrain, 0)
        plsc.subcore_barrier()

        def load(k, b):
            pltpu.async_copy(vals_hbm.at[pl.ds((wid * R + k) * d, d)], vb[b],
                             lsem)

        load(0, 0)
        load(1, 1)

        def round_(r, carry):
            for i in range(2):
                k = r * 2 + i
                pltpu.make_async_copy(dummy, vb[i], lsem).wait()
                pltpu.sync_copy(vb[i], table_sh.at[idx_v.at[k]], add=True)
                load(k + 2, i)
            return carry

        lax.fori_loop(0, R // 2 - 1, round_, 0)
        for i in range(2):
            k = R - 2 + i
            pltpu.make_async_copy(dummy, vb[i], lsem).wait()
            pltpu.sync_copy(vb[i], table_sh.at[idx_v.at[k]], add=True)
        plsc.subcore_barrier()

        for m in range(ZR // c_sz):
            off = sid * ZR + m * c_sz
            pltpu.sync_copy(table_sh.at[pl.ds(off, c_sz)], v0)
            pltpu.sync_copy(v0, out_hbm.at[cid, pl.ds(off, c_sz)])

    f = pl.kernel(
        body,
        out_type=jax.ShapeDtypeStruct((_NC, TBL, C), jnp.float32),
        mesh=mesh,
        scratch_types=[
            pltpu.VMEM((R, d), jnp.int32),
            pltpu.VMEM((d, C), jnp.float32),
            pltpu.VMEM((d, C), jnp.float32),
            pltpu.VMEM((16, C), jnp.float32),
            pltpu.VMEM_SHARED((TBL, C), jnp.float32),
            pltpu.SemaphoreType.DMA,
        ],
    )
    return f(vals, idx2d)


def _round_to_f16(x):
    u = lax.bitcast_convert_type(x, jnp.uint32)
    lsb = (u >> jnp.uint32(13)) & jnp.uint32(1)
    u = (u + jnp.uint32(0x0FFF) + lsb) & jnp.uint32(0xFFFFE000)
    return lax.bitcast_convert_type(u, jnp.float32)


def _tc_edge(edge2, gath2, g1, b1, W1p, b1p, W2p, b2p, e_pad_rows):
    N_e, C = edge2.shape
    E = 2000
    grid = (N_e // E,)

    def body(ed_ref, gt_ref, g_ref, bb_ref, w1_ref, c1_ref, w2_ref, c2_ref,
             e32_ref, eo_ref):
        ed = ed_ref[...]
        gt = gt_ref[...]
        inv_n = 1.0 / (3 * C)
        mu = (jnp.sum(ed, axis=1, keepdims=True)
              + jnp.sum(gt, axis=1, keepdims=True)) * inv_n
        d1 = ed - mu
        d2 = gt - mu
        var = (jnp.sum(d1 * d1, axis=1, keepdims=True)
               + jnp.sum(d2 * d2, axis=1, keepdims=True)) * inv_n
        s = 1.0 / jnp.sqrt(var + 1e-5)
        x = jnp.concatenate([d1, d2], axis=1) * s * g_ref[...] + bb_ref[...]
        h = jnp.dot(x, w1_ref[...], preferred_element_type=jnp.float32) + c1_ref[...]
        h = 0.5 * h * (1.0 + lax.erf(h * 0.7071067811865476))
        e = jnp.dot(h, w2_ref[...], preferred_element_type=jnp.float32) + c2_ref[...]
        e32 = _round_to_f16(e)
        e32_ref[...] = e32
        eo_ref[...] = ed + e32

    return pl.pallas_call(
        body,
        grid=grid,
        in_specs=[
            pl.BlockSpec((E, C), lambda i: (i, 0)),
            pl.BlockSpec((E, 2 * C), lambda i: (i, 0)),
            pl.BlockSpec((1, 3 * C), lambda i: (0, 0)),
            pl.BlockSpec((1, 3 * C), lambda i: (0, 0)),
            pl.BlockSpec((3 * C, C), lambda i: (0, 0)),
            pl.BlockSpec((1, C), lambda i: (0, 0)),
            pl.BlockSpec((C, C), lambda i: (0, 0)),
            pl.BlockSpec((1, C), lambda i: (0, 0)),
        ],
        out_specs=[
            pl.BlockSpec((E, C), lambda i: (i, 0)),
            pl.BlockSpec((E, C), lambda i: (i, 0)),
        ],
        out_shape=[
            jax.ShapeDtypeStruct((e_pad_rows, C), jnp.float32),
            jax.ShapeDtypeStruct((N_e, C), jnp.float32),
        ],
    )(edge2, gath2, g1, b1, W1p, b1p, W2p, b2p)


def _tc_node(node2, aggs, g2, b2, W1p, b1p, W2p, b2p):
    N_n, C = node2.shape
    NB = 1000
    grid = (N_n // NB,)

    def body(nd_ref, a0_ref, a1_ref, g_ref, bb_ref, w1_ref, c1_ref, w2_ref,
             c2_ref, out_ref):
        nd = nd_ref[...]
        ag = a0_ref[0] + a1_ref[0]
        inv_n = 1.0 / (2 * C)
        mu = (jnp.sum(nd, axis=1, keepdims=True)
              + jnp.sum(ag, axis=1, keepdims=True)) * inv_n
        d1 = nd - mu
        d2 = ag - mu
        var = (jnp.sum(d1 * d1, axis=1, keepdims=True)
               + jnp.sum(d2 * d2, axis=1, keepdims=True)) * inv_n
        s = 1.0 / jnp.sqrt(var + 1e-5)
        x = jnp.concatenate([d1, d2], axis=1) * s * g_ref[...] + bb_ref[...]
        h = jnp.dot(x, w1_ref[...], preferred_element_type=jnp.float32) + c1_ref[...]
        h = 0.5 * h * (1.0 + lax.erf(h * 0.7071067811865476))
        o = jnp.dot(h, w2_ref[...], preferred_element_type=jnp.float32) + c2_ref[...]
        out_ref[...] = nd + o

    return pl.pallas_call(
        body,
        grid=grid,
        in_specs=[
            pl.BlockSpec((NB, C), lambda i: (i, 0)),
            pl.BlockSpec((1, NB, C), lambda i: (0, i, 0)),
            pl.BlockSpec((1, NB, C), lambda i: (1, i, 0)),
            pl.BlockSpec((1, 2 * C), lambda i: (0, 0)),
            pl.BlockSpec((1, 2 * C), lambda i: (0, 0)),
            pl.BlockSpec((2 * C, C), lambda i: (0, 0)),
            pl.BlockSpec((1, C), lambda i: (0, 0)),
            pl.BlockSpec((C, C), lambda i: (0, 0)),
            pl.BlockSpec((1, C), lambda i: (0, 0)),
        ],
        out_specs=pl.BlockSpec((NB, C), lambda i: (i, 0)),
        out_shape=jax.ShapeDtypeStruct((N_n, C), jnp.float32),
    )(node2, aggs, aggs, g2, b2, W1p, b1p, W2p, b2p)


def kernel(node, edge, edgeIdx, edge2node, g1, b1, g2, b2,
           We1, be1, We2, be2, Wn1, bn1, Wn2, bn2):
    B, N_n, C = node.shape
    N_e = edge.shape[1]
    H3 = We1.shape[1]
    H2 = Wn1.shape[1]

    node2 = node.reshape(N_n, C)
    edge2 = edge.reshape(N_e, C)

    nf = 2 * N_e
    Rg = ((-(-nf // (_NW * _D)) + 7) // 8) * 8
    pad_g = _NW * _D * Rg - nf
    idx_flat = edgeIdx.astype(jnp.int32).reshape(-1)
    idx2d = jnp.concatenate(
        [idx_flat, jnp.zeros((pad_g,), jnp.int32)]).reshape(_NW * Rg, _D)
    gath = _sc_gather(node2, idx2d)
    gath2 = gath.reshape(-1, 2 * C)

    W1p = jnp.pad(We1, ((0, 0), (0, C - H3)))
    b1p = jnp.pad(be1, (0, C - H3)).reshape(1, C)
    W2p = jnp.pad(We2, ((0, C - H3), (0, 0)))
    b2p = be2.reshape(1, C)

    ns = N_e // _D
    Rs = ((-(-ns // _NW) + 7) // 8) * 8
    e_pad_rows = _NW * Rs * _D
    TBL = ((N_n + _NS * 16 - 1) // (_NS * 16)) * (_NS * 16)

    e32, edge_out = _tc_edge(edge2, gath2, g1.reshape(1, 3 * C),
                             b1.reshape(1, 3 * C), W1p, b1p, W2p, b2p,
                             e_pad_rows)

    pad_s = e_pad_rows - N_e
    e2n = jnp.concatenate(
        [edge2node.astype(jnp.int32),
         jnp.full((pad_s,), TBL - 1, jnp.int32)]).reshape(_NW * Rs, _D)
    aggs = _sc_scatter(e32, e2n, N_n)

    V1p = jnp.pad(Wn1, ((0, 0), (0, C - H2)))
    d1p = jnp.pad(bn1, (0, C - H2)).reshape(1, C)
    V2p = jnp.pad(Wn2, ((0, C - H2), (0, 0)))
    d2p = bn2.reshape(1, C)

    node_out = _tc_node(node2, aggs, g2.reshape(1, 2 * C),
                        b2.reshape(1, 2 * C), V1p, d1p, V2p, d2p)

    return (node_out.reshape(B, N_n, C), edge_out.reshape(B, N_e, C))
